# Initial kernel scaffold; baseline (speedup 1.0000x reference)
import jax
import jax.numpy as jnp
from jax.experimental import pallas as pl


def kernel(inputs, edge_index, W1, b1, W2, b2):
    raise NotImplementedError("write your pallas kernel here")


# speedup vs baseline: 14.8994x; 14.8994x over previous
import functools

import jax
import jax.numpy as jnp
from jax import lax
from jax.experimental import pallas as pl
from jax.experimental.pallas import tpu as pltpu
from jax.experimental.pallas import tpu_sc as plsc

N_CORES = 2
N_SUBCORES = 16
N_WORKERS = N_CORES * N_SUBCORES


def _make_deg_kernel(E, NP):
    EPW = E // N_WORKERS
    CH = 80
    assert EPW % CH == 0
    NCH = EPW // CH
    RPT = NP // N_SUBCORES

    mesh = plsc.VectorSubcoreMesh(core_axis_name="c", subcore_axis_name="s")

    @functools.partial(
        pl.kernel,
        out_type=jax.ShapeDtypeStruct((N_CORES, NP, 16), jnp.float32),
        mesh=mesh,
        scratch_types=[
            pltpu.VMEM_SHARED((NP, 16), jnp.float32),
            pltpu.VMEM((CH, 16), jnp.float32),
            pltpu.VMEM((CH,), jnp.int32),
        ],
    )
    def deg_kernel(dst_hbm, zeros_hbm, ones_hbm, out_hbm, acc_sh, ones_v, idx_v):
        c = lax.axis_index("c")
        s = lax.axis_index("s")
        wid = s * N_CORES + c
        pltpu.sync_copy(ones_hbm, ones_v)
        pltpu.sync_copy(zeros_hbm, acc_sh.at[pl.ds(s * RPT, RPT)])
        plsc.subcore_barrier()
        base = wid * EPW

        def body(k, carry):
            pltpu.sync_copy(dst_hbm.at[pl.ds(base + k * CH, CH)], idx_v)
            pltpu.sync_copy(ones_v, acc_sh.at[idx_v], add=True)
            return carry

        lax.fori_loop(0, NCH, body, 0)
        plsc.subcore_barrier()
        pltpu.sync_copy(acc_sh.at[pl.ds(s * RPT, RPT)],
                        out_hbm.at[c, pl.ds(s * RPT, RPT)])

    return deg_kernel


def _make_agg_kernel(E, NP, H):
    EPW = E // N_WORKERS
    CH = 128
    NCH = EPW // CH
    REM = EPW - NCH * CH
    assert REM % 8 == 0
    RPT = NP // N_SUBCORES
    ZCH = 128
    assert RPT % ZCH == 0

    mesh = plsc.VectorSubcoreMesh(core_axis_name="c", subcore_axis_name="s")

    scratch = [
        pltpu.VMEM_SHARED((NP, H), jnp.float32),
        pltpu.VMEM((CH, H), jnp.float32),
        pltpu.VMEM((CH,), jnp.int32),
        pltpu.VMEM((CH,), jnp.int32),
        pltpu.SemaphoreType.DMA,
    ]
    if REM:
        scratch += [
            pltpu.VMEM((REM, H), jnp.float32),
            pltpu.VMEM((REM,), jnp.int32),
            pltpu.VMEM((REM,), jnp.int32),
        ]

    @functools.partial(
        pl.kernel,
        out_type=jax.ShapeDtypeStruct((N_CORES, NP, H), jnp.float32),
        mesh=mesh,
        scratch_types=scratch,
    )
    def agg_kernel(y_hbm, src_hbm, dst_hbm, zeros_hbm, out_hbm,
                   acc_sh, rows_v, sidx_v, didx_v, sem, *rem_scratch):
        c = lax.axis_index("c")
        s = lax.axis_index("s")
        wid = s * N_CORES + c
        for z in range(RPT // ZCH):
            pltpu.sync_copy(zeros_hbm,
                            acc_sh.at[pl.ds(s * RPT + z * ZCH, ZCH)])
        plsc.subcore_barrier()
        base = wid * EPW

        def body(k, carry):
            off = base + k * CH
            pltpu.sync_copy(src_hbm.at[pl.ds(off, CH)], sidx_v)
            pltpu.sync_copy(dst_hbm.at[pl.ds(off, CH)], didx_v)
            pltpu.async_copy(y_hbm.at[sidx_v], rows_v, sem).wait()
            pltpu.sync_copy(rows_v, acc_sh.at[didx_v], add=True)
            return carry

        lax.fori_loop(0, NCH, body, 0)
        if REM:
            rows_r, sidx_r, didx_r = rem_scratch
            off = base + NCH * CH
            pltpu.sync_copy(src_hbm.at[pl.ds(off, REM)], sidx_r)
            pltpu.sync_copy(dst_hbm.at[pl.ds(off, REM)], didx_r)
            pltpu.async_copy(y_hbm.at[sidx_r], rows_r, sem).wait()
            pltpu.sync_copy(rows_r, acc_sh.at[didx_r], add=True)
        plsc.subcore_barrier()
        pltpu.sync_copy(acc_sh.at[pl.ds(s * RPT, RPT)],
                        out_hbm.at[c, pl.ds(s * RPT, RPT)])

    return agg_kernel


def _tc1_body(deg_ref, x_ref, w_ref, y_ref, dis_ref):
    d0 = deg_ref[0, :, 0:1]
    d1 = deg_ref[1, :, 0:1]
    dis = lax.rsqrt(d0 + d1 + 1.0)
    y_ref[...] = jnp.dot(x_ref[...], w_ref[...],
                         preferred_element_type=jnp.float32) * dis
    dis_ref[...] = jnp.broadcast_to(dis, dis_ref.shape)


def _tc2_body(agg_ref, y_ref, dis_ref, b_ref, w_ref, o_ref):
    h = (agg_ref[0] + agg_ref[1] + y_ref[...]) * dis_ref[...] + b_ref[...]
    h = jnp.maximum(h, 0.0)
    o_ref[...] = jnp.dot(h, w_ref[...],
                         preferred_element_type=jnp.float32) * dis_ref[...]


def _tc3_body(agg_ref, y_ref, dis_ref, b_ref, o_ref):
    h = (agg_ref[0] + agg_ref[1] + y_ref[...]) * dis_ref[...] + b_ref[...]
    o_ref[...] = jnp.maximum(h, 0.0)


def kernel(inputs, edge_index, W1, b1, W2, b2):
    N, F = inputs.shape
    H = W1.shape[1]
    E = edge_index.shape[1]
    assert E % N_WORKERS == 0 and (E // N_WORKERS) % 8 == 0

    B = 640
    NP = -(-N // (N_SUBCORES * B)) * (N_SUBCORES * B)
    G = NP // B

    src = edge_index[0]
    dst = edge_index[1]
    x = jnp.pad(inputs, ((0, NP - N), (0, 0)))

    zeros_deg = jnp.zeros((NP // N_SUBCORES, 16), jnp.float32)
    ones_deg = jnp.ones((80, 16), jnp.float32)
    zeros_seg = jnp.zeros((128, H), jnp.float32)

    degp = _make_deg_kernel(E, NP)(dst, zeros_deg, ones_deg)

    tc1 = pl.pallas_call(
        _tc1_body,
        grid=(G,),
        in_specs=[
            pl.BlockSpec((2, B, 16), lambda i: (0, i, 0)),
            pl.BlockSpec((B, F), lambda i: (i, 0)),
            pl.BlockSpec((F, H), lambda i: (0, 0)),
        ],
        out_specs=[
            pl.BlockSpec((B, H), lambda i: (i, 0)),
            pl.BlockSpec((B, H), lambda i: (i, 0)),
        ],
        out_shape=[
            jax.ShapeDtypeStruct((NP, H), jnp.float32),
            jax.ShapeDtypeStruct((NP, H), jnp.float32),
        ],
    )
    y1, dis = tc1(degp, x, W1)

    agg_fn = _make_agg_kernel(E, NP, H)
    agg1 = agg_fn(y1, src, dst, zeros_seg)

    tc2 = pl.pallas_call(
        _tc2_body,
        grid=(G,),
        in_specs=[
            pl.BlockSpec((2, B, H), lambda i: (0, i, 0)),
            pl.BlockSpec((B, H), lambda i: (i, 0)),
            pl.BlockSpec((B, H), lambda i: (i, 0)),
            pl.BlockSpec((1, H), lambda i: (0, 0)),
            pl.BlockSpec((H, H), lambda i: (0, 0)),
        ],
        out_specs=pl.BlockSpec((B, H), lambda i: (i, 0)),
        out_shape=jax.ShapeDtypeStruct((NP, H), jnp.float32),
    )
    y2 = tc2(agg1, y1, dis, b1.reshape(1, H), W2)

    agg2 = agg_fn(y2, src, dst, zeros_seg)

    tc3 = pl.pallas_call(
        _tc3_body,
        grid=(G,),
        in_specs=[
            pl.BlockSpec((2, B, H), lambda i: (0, i, 0)),
            pl.BlockSpec((B, H), lambda i: (i, 0)),
            pl.BlockSpec((B, H), lambda i: (i, 0)),
            pl.BlockSpec((1, H), lambda i: (0, 0)),
        ],
        out_specs=pl.BlockSpec((B, H), lambda i: (i, 0)),
        out_shape=jax.ShapeDtypeStruct((NP, H), jnp.float32),
    )
    out = tc3(agg2, y2, dis, b2.reshape(1, H))
    return out[:N]
